# Initial kernel scaffold; baseline (speedup 1.0000x reference)
import jax
import jax.numpy as jnp
from jax.experimental import pallas as pl


def kernel(x, W):
    raise NotImplementedError("write your pallas kernel here")


# speedup vs baseline: 8.0819x; 8.0819x over previous
import jax
import jax.numpy as jnp
from jax.experimental import pallas as pl
from jax.experimental.pallas import tpu as pltpu

ROWS = 16384
SEQ = 200
EMB = 64
VOCAB = 8
BLOCK_R = 128


def _body(x_ref, w_ref, o_ref):
    x = x_ref[...]
    w = w_ref[...]
    oh = (x[:, :, None] == jax.lax.broadcasted_iota(jnp.int32, (1, 1, VOCAB), 2))
    oh = oh.astype(jnp.float32).reshape(BLOCK_R * SEQ, VOCAB)
    rows = jax.lax.dot_general(
        oh, w, (((1,), (0,)), ((), ())), preferred_element_type=jnp.float32
    )
    o_ref[...] = rows.reshape(BLOCK_R, SEQ, EMB)


def kernel(x, W):
    grid = (ROWS // BLOCK_R,)
    return pl.pallas_call(
        _body,
        grid=grid,
        in_specs=[
            pl.BlockSpec((BLOCK_R, SEQ), lambda i: (i, 0)),
            pl.BlockSpec((VOCAB, EMB), lambda i: (0, 0)),
        ],
        out_specs=pl.BlockSpec((BLOCK_R, SEQ, EMB), lambda i: (i, 0, 0)),
        out_shape=jax.ShapeDtypeStruct((ROWS, SEQ, EMB), jnp.float32),
    )(x, W)
